# baseline (speedup 1.0000x reference)
import jax
import jax.numpy as jnp
from jax.experimental import pallas as pl
from jax.experimental.pallas import tpu as pltpu

N = 16384
D = 2048
BLK = 512


def _body(t_ref, s_ref, xp_ref, x_ref, o_ref):
    d = xp_ref[...] - x_ref[...]
    ssq = jnp.sum(d * d, axis=1)
    s = s_ref[0, 0]
    logs = jnp.log(s)
    scale = -logs * jnp.exp(-2.0 * logs * t_ref[:, 0])
    o_ref[...] = scale * ssq


def kernel(t, sigma1, x_pred, x):
    n, d = x.shape
    s2d = sigma1.reshape(1, 1)
    grid = (n // BLK,)
    out = pl.pallas_call(
        _body,
        grid=grid,
        in_specs=[
            pl.BlockSpec((BLK, 1), lambda i: (i, 0)),
            pl.BlockSpec((1, 1), lambda i: (0, 0)),
            pl.BlockSpec((BLK, d), lambda i: (i, 0)),
            pl.BlockSpec((BLK, d), lambda i: (i, 0)),
        ],
        out_specs=pl.BlockSpec((BLK,), lambda i: (i,)),
        out_shape=jax.ShapeDtypeStruct((n,), jnp.float32),
        compiler_params=pltpu.CompilerParams(
            dimension_semantics=("arbitrary",),
        ),
    )(t, s2d, x_pred, x)
    return out
